# Initial kernel scaffold; baseline (speedup 1.0000x reference)
import jax
import jax.numpy as jnp
from jax.experimental import pallas as pl


def kernel(x, edge_index, batch, W1, b1, W2, b2, W3, b3):
    raise NotImplementedError("write your pallas kernel here")


# speedup vs baseline: 9.0295x; 9.0295x over previous
import functools

import jax
import jax.numpy as jnp
from jax import lax
from jax.experimental import pallas as pl
from jax.experimental.pallas import tpu as pltpu
from jax.experimental.pallas import tpu_sc as plsc

NC = 2
NS = 16
NW = NC * NS
L = 128
LANES = 16
HW = 16
NACC = 10112
ROWS_PER = NACC // NS


def _mesh():
    return plsc.VectorSubcoreMesh(
        core_axis_name="c", subcore_axis_name="s", num_cores=NC, num_subcores=NS
    )


def _unpack_chunk(packed_v, j, idxbuf, want_src):
    for r in range(L // LANES):
        pv = packed_v[j, pl.ds(LANES * r, LANES)]
        if want_src:
            idxbuf[0, pl.ds(LANES * r, LANES)] = lax.shift_right_logical(pv, 16)
        idxbuf[1, pl.ds(LANES * r, LANES)] = pv & 0xFFFF


def _make_hist(ch, chp, f):
    @functools.partial(
        pl.kernel,
        out_type=jax.ShapeDtypeStruct((NC * NACC, f), jnp.float32),
        mesh=_mesh(),
        scratch_types=[
            pltpu.VMEM((chp, L), jnp.int32),
            pltpu.VMEM((8, L), jnp.int32),
            pltpu.VMEM((L, f), jnp.float32),
            pltpu.VMEM_SHARED((NACC, f), jnp.float32),
        ],
    )
    def hist(packed_hbm, ones_hbm, zeros_hbm, out_hbm, packed_v, idxbuf, ones_v, acc_sh):
        cid = lax.axis_index("c")
        sid = lax.axis_index("s")
        wid = sid * NC + cid
        pltpu.sync_copy(
            zeros_hbm.at[pl.ds(sid * ROWS_PER, ROWS_PER)],
            acc_sh.at[pl.ds(sid * ROWS_PER, ROWS_PER)],
        )
        pltpu.sync_copy(ones_hbm, ones_v)
        pltpu.sync_copy(packed_hbm.at[wid], packed_v)
        plsc.subcore_barrier()

        def body(j, c):
            _unpack_chunk(packed_v, j, idxbuf, want_src=False)
            pltpu.sync_copy(ones_v, acc_sh.at[idxbuf.at[1]], add=True)
            return c

        lax.fori_loop(0, ch, body, 0)
        plsc.subcore_barrier()
        pltpu.sync_copy(
            acc_sh.at[pl.ds(sid * ROWS_PER, ROWS_PER)],
            out_hbm.at[pl.ds(cid * NACC + sid * ROWS_PER, ROWS_PER)],
        )

    return hist


def _make_prop(ch, chp, f):
    @functools.partial(
        pl.kernel,
        out_type=jax.ShapeDtypeStruct((NC * NACC, f), jnp.float32),
        mesh=_mesh(),
        scratch_types=[
            pltpu.VMEM((chp, L), jnp.int32),
            pltpu.VMEM((8, L), jnp.int32),
            pltpu.VMEM((8, L), jnp.int32),
            pltpu.VMEM((L, f), jnp.float32),
            pltpu.VMEM((L, f), jnp.float32),
            pltpu.VMEM_SHARED((NACC, f), jnp.float32),
            pltpu.SemaphoreType.DMA,
            pltpu.SemaphoreType.DMA,
        ],
    )
    def prop(hs_hbm, packed_hbm, zeros_hbm, out_hbm,
             packed_v, idx0, idx1, buf0, buf1, acc_sh, sem0, sem1):
        cid = lax.axis_index("c")
        sid = lax.axis_index("s")
        wid = sid * NC + cid
        pltpu.sync_copy(
            zeros_hbm.at[pl.ds(sid * ROWS_PER, ROWS_PER)],
            acc_sh.at[pl.ds(sid * ROWS_PER, ROWS_PER)],
        )
        pltpu.sync_copy(packed_hbm.at[wid], packed_v)
        plsc.subcore_barrier()

        _unpack_chunk(packed_v, 0, idx0, want_src=True)
        _unpack_chunk(packed_v, 1, idx1, want_src=True)
        pltpu.async_copy(hs_hbm.at[idx0.at[0]], buf0, sem0)
        pltpu.async_copy(hs_hbm.at[idx1.at[0]], buf1, sem1)

        def body(i, c):
            j = 2 * i
            pltpu.make_async_copy(hs_hbm.at[idx0.at[0]], buf0, sem0).wait()
            pltpu.sync_copy(buf0, acc_sh.at[idx0.at[1]], add=True)

            @pl.when(j + 2 < ch)
            def _issue0():
                _unpack_chunk(packed_v, j + 2, idx0, want_src=True)
                pltpu.async_copy(hs_hbm.at[idx0.at[0]], buf0, sem0)

            pltpu.make_async_copy(hs_hbm.at[idx1.at[0]], buf1, sem1).wait()
            pltpu.sync_copy(buf1, acc_sh.at[idx1.at[1]], add=True)

            @pl.when(j + 3 < ch)
            def _issue1():
                _unpack_chunk(packed_v, j + 3, idx1, want_src=True)
                pltpu.async_copy(hs_hbm.at[idx1.at[0]], buf1, sem1)

            return c

        lax.fori_loop(0, ch // 2, body, 0)
        plsc.subcore_barrier()
        pltpu.sync_copy(
            acc_sh.at[pl.ds(sid * ROWS_PER, ROWS_PER)],
            out_hbm.at[pl.ds(cid * NACC + sid * ROWS_PER, ROWS_PER)],
        )

    return prop


_BR = 1000


def _tc_first(deg0, deg1, x, w):
    n, d = x.shape
    h = w.shape[1]

    def body(d0, d1, xr, wr, hs_ref, dinv_ref):
        deg = d0[:, :HW] + d1[:, :HW]
        dinv = jnp.where(deg > 0, lax.rsqrt(deg), 0.0)
        dinv_ref[...] = dinv
        hh = jnp.dot(xr[...], wr[...], preferred_element_type=jnp.float32)
        hs_ref[...] = hh * dinv[:, :1]

    return pl.pallas_call(
        body,
        grid=(n // _BR,),
        in_specs=[
            pl.BlockSpec((_BR, deg0.shape[1]), lambda i: (i, 0)),
            pl.BlockSpec((_BR, deg1.shape[1]), lambda i: (i, 0)),
            pl.BlockSpec((_BR, d), lambda i: (i, 0)),
            pl.BlockSpec((d, h), lambda i: (0, 0)),
        ],
        out_specs=[
            pl.BlockSpec((_BR, h), lambda i: (i, 0)),
            pl.BlockSpec((_BR, HW), lambda i: (i, 0)),
        ],
        out_shape=[
            jax.ShapeDtypeStruct((n, h), jnp.float32),
            jax.ShapeDtypeStruct((n, HW), jnp.float32),
        ],
    )(deg0, deg1, x, w)


def _tc_mid(p0, p1, dinv, b, w):
    n, d = p0.shape
    h = w.shape[1]

    def body(p0r, p1r, dvr, br, wr, out_ref):
        dv = dvr[:, :1]
        t = jnp.maximum((p0r[...] + p1r[...]) * dv + br[...], 0.0)
        out_ref[...] = jnp.dot(t, wr[...], preferred_element_type=jnp.float32) * dv

    return pl.pallas_call(
        body,
        grid=(n // _BR,),
        in_specs=[
            pl.BlockSpec((_BR, d), lambda i: (i, 0)),
            pl.BlockSpec((_BR, d), lambda i: (i, 0)),
            pl.BlockSpec((_BR, HW), lambda i: (i, 0)),
            pl.BlockSpec((1, d), lambda i: (0, 0)),
            pl.BlockSpec((d, h), lambda i: (0, 0)),
        ],
        out_specs=pl.BlockSpec((_BR, h), lambda i: (i, 0)),
        out_shape=jax.ShapeDtypeStruct((n, h), jnp.float32),
    )(p0, p1, dinv, b, w)


def _tc_scale(p0, p1, dinv, b):
    n, d = p0.shape

    def body(p0r, p1r, dvr, br, out_ref):
        dv = dvr[:, :1]
        out_ref[...] = jnp.maximum((p0r[...] + p1r[...]) * dv + br[...], 0.0) * dv

    return pl.pallas_call(
        body,
        grid=(n // _BR,),
        in_specs=[
            pl.BlockSpec((_BR, d), lambda i: (i, 0)),
            pl.BlockSpec((_BR, d), lambda i: (i, 0)),
            pl.BlockSpec((_BR, HW), lambda i: (i, 0)),
            pl.BlockSpec((1, d), lambda i: (0, 0)),
        ],
        out_specs=pl.BlockSpec((_BR, d), lambda i: (i, 0)),
        out_shape=jax.ShapeDtypeStruct((n, d), jnp.float32),
    )(p0, p1, dinv, b)


def _tc_final(p0, p1, dinv, w, b):
    n, d = p0.shape
    c = w.shape[1]

    def body(p0r, p1r, dvr, wr, br, out_ref):
        q = jnp.dot(p0r[...] + p1r[...], wr[...], preferred_element_type=jnp.float32)
        out_ref[...] = q * dvr[:, :1] + br[...]

    return pl.pallas_call(
        body,
        grid=(n // _BR,),
        in_specs=[
            pl.BlockSpec((_BR, d), lambda i: (i, 0)),
            pl.BlockSpec((_BR, d), lambda i: (i, 0)),
            pl.BlockSpec((_BR, HW), lambda i: (i, 0)),
            pl.BlockSpec((d, c), lambda i: (0, 0)),
            pl.BlockSpec((1, c), lambda i: (0, 0)),
        ],
        out_specs=pl.BlockSpec((_BR, c), lambda i: (i, 0)),
        out_shape=jax.ShapeDtypeStruct((n, c), jnp.float32),
    )(p0, p1, dinv, w, b)


def kernel(x, edge_index, batch, W1, b1, W2, b2, W3, b3):
    n, d = x.shape
    e = edge_index.shape[1]
    h = W1.shape[1]
    c = W3.shape[1]

    ch = -(-(e + n) // (NW * L))
    ch += ch % 2
    chp = -(-ch // 8) * 8
    pad = NW * ch * L - e - n
    loop = jnp.arange(n, dtype=jnp.int32)
    src = jnp.concatenate([edge_index[0], loop, jnp.zeros((pad,), jnp.int32)])
    dst = jnp.concatenate([edge_index[1], loop, jnp.full((pad,), n, jnp.int32)])
    packed = (src << 16) | dst
    packed = jnp.concatenate(
        [packed.reshape(NW, ch, L),
         jnp.zeros((NW, chp - ch, L), jnp.int32)], axis=1)

    onesf = jnp.ones((L, h), jnp.float32)
    zf = jnp.zeros((NACC, h), jnp.float32)

    deg = _make_hist(ch, chp, h)(packed, onesf, zf)
    hs1, dinv = _tc_first(deg[0:n], deg[NACC:NACC + n], x, W1)

    prop_h = _make_prop(ch, chp, h)
    p = prop_h(hs1, packed, zf)
    hs2 = _tc_mid(p[0:n], p[NACC:NACC + n], dinv, b1.reshape(1, -1), W2)

    p = prop_h(hs2, packed, zf)
    ts3 = _tc_scale(p[0:n], p[NACC:NACC + n], dinv, b2.reshape(1, -1))

    p3 = prop_h(ts3, packed, zf)
    return _tc_final(p3[0:n], p3[NACC:NACC + n], dinv, W3, b3.reshape(1, -1))


# speedup vs baseline: 16.7430x; 1.8543x over previous
import functools

import jax
import jax.numpy as jnp
from jax import lax
from jax.experimental import pallas as pl
from jax.experimental.pallas import tpu as pltpu
from jax.experimental.pallas import tpu_sc as plsc

NC = 2
NS = 16
NW = NC * NS
L = 128
LANES = 16
HW = 16
NACC = 10112
ROWS_PER = NACC // NS

CH0 = 122
CH1 = 40
CH1P = 40


def _mesh():
    return plsc.VectorSubcoreMesh(
        core_axis_name="c", subcore_axis_name="s", num_cores=NC, num_subcores=NS
    )


def _unpack_chunk(packed_v, j, idxbuf, base, want_src):
    for r in range(L // LANES):
        pv = packed_v[j, pl.ds(LANES * r, LANES)]
        if want_src:
            idxbuf[base, pl.ds(LANES * r, LANES)] = lax.shift_right_logical(pv, 16)
        idxbuf[base + 1, pl.ds(LANES * r, LANES)] = pv & 0xFFFF


def _stage(packed_hbm, packed_v, cid, widx):
    @pl.when(cid == 0)
    def _full():
        pltpu.sync_copy(packed_hbm.at[widx], packed_v)

    @pl.when(cid == 1)
    def _part():
        pltpu.sync_copy(packed_hbm.at[widx, pl.ds(0, CH1P)],
                        packed_v.at[pl.ds(0, CH1P)])


def _make_hist(f):
    @functools.partial(
        pl.kernel,
        out_type=jax.ShapeDtypeStruct((NC * NACC, f), jnp.float32),
        mesh=_mesh(),
        scratch_types=[
            pltpu.VMEM((CH0, L), jnp.int32),
            pltpu.VMEM((8, L), jnp.int32),
            pltpu.VMEM((L, f), jnp.float32),
            pltpu.VMEM_SHARED((NACC, f), jnp.float32),
        ],
    )
    def hist(packed_hbm, ones_hbm, zeros_hbm, out_hbm, packed_v, idxbuf, ones_v, acc_sh):
        cid = lax.axis_index("c")
        sid = lax.axis_index("s")
        widx = cid * NS + sid
        pltpu.sync_copy(
            zeros_hbm.at[pl.ds(sid * ROWS_PER, ROWS_PER)],
            acc_sh.at[pl.ds(sid * ROWS_PER, ROWS_PER)],
        )
        pltpu.sync_copy(ones_hbm, ones_v)
        _stage(packed_hbm, packed_v, cid, widx)
        plsc.subcore_barrier()
        nch = jnp.where(cid == 0, CH0, CH1)

        def body(j, c):
            _unpack_chunk(packed_v, j, idxbuf, 0, want_src=False)
            pltpu.sync_copy(ones_v, acc_sh.at[idxbuf.at[1]], add=True)
            return c

        lax.fori_loop(0, nch, body, 0)
        plsc.subcore_barrier()
        pltpu.sync_copy(
            acc_sh.at[pl.ds(sid * ROWS_PER, ROWS_PER)],
            out_hbm.at[pl.ds(cid * NACC + sid * ROWS_PER, ROWS_PER)],
        )

    return hist


def _make_prop(f):
    @functools.partial(
        pl.kernel,
        out_type=jax.ShapeDtypeStruct((NC * NACC, f), jnp.float32),
        mesh=_mesh(),
        scratch_types=[
            pltpu.VMEM((CH0, L), jnp.int32),
            pltpu.VMEM((8, L), jnp.int32),
            pltpu.VMEM((L, f), jnp.float32),
            pltpu.VMEM((L, f), jnp.float32),
            pltpu.VMEM_SHARED((NACC, f), jnp.float32),
            pltpu.SemaphoreType.DMA,
            pltpu.SemaphoreType.DMA,
        ],
    )
    def prop(hs_hbm, packed_hbm, zeros_hbm, out_hbm,
             packed_v, idxb, buf0, buf1, acc_sh, sem0, sem1):
        cid = lax.axis_index("c")
        sid = lax.axis_index("s")
        widx = cid * NS + sid
        pltpu.sync_copy(
            zeros_hbm.at[pl.ds(sid * ROWS_PER, ROWS_PER)],
            acc_sh.at[pl.ds(sid * ROWS_PER, ROWS_PER)],
        )
        _stage(packed_hbm, packed_v, cid, widx)
        plsc.subcore_barrier()
        nch = jnp.where(cid == 0, CH0, CH1)

        _unpack_chunk(packed_v, 0, idxb, 0, want_src=True)
        _unpack_chunk(packed_v, 1, idxb, 2, want_src=True)
        pltpu.async_copy(hs_hbm.at[idxb.at[0]], buf0, sem0)
        pltpu.async_copy(hs_hbm.at[idxb.at[2]], buf1, sem1)

        def body(i, c):
            j = 2 * i
            pltpu.make_async_copy(hs_hbm.at[idxb.at[0]], buf0, sem0).wait()
            pltpu.sync_copy(buf0, acc_sh.at[idxb.at[1]], add=True)

            @pl.when(j + 2 < nch)
            def _issue0():
                _unpack_chunk(packed_v, j + 2, idxb, 0, want_src=True)
                pltpu.async_copy(hs_hbm.at[idxb.at[0]], buf0, sem0)

            pltpu.make_async_copy(hs_hbm.at[idxb.at[2]], buf1, sem1).wait()
            pltpu.sync_copy(buf1, acc_sh.at[idxb.at[3]], add=True)

            @pl.when(j + 3 < nch)
            def _issue1():
                _unpack_chunk(packed_v, j + 3, idxb, 2, want_src=True)
                pltpu.async_copy(hs_hbm.at[idxb.at[2]], buf1, sem1)

            return c

        lax.fori_loop(0, nch // 2, body, 0)
        plsc.subcore_barrier()
        pltpu.sync_copy(
            acc_sh.at[pl.ds(sid * ROWS_PER, ROWS_PER)],
            out_hbm.at[pl.ds(cid * NACC + sid * ROWS_PER, ROWS_PER)],
        )

    return prop


_BR = 1000


def _tc_first(deg0, deg1, x, w):
    n, d = x.shape
    h = w.shape[1]

    def body(d0, d1, xr, wr, hs_ref, dinv_ref):
        deg = d0[:, :HW] + d1[:, :HW]
        dinv = jnp.where(deg > 0, lax.rsqrt(deg), 0.0)
        dinv_ref[...] = dinv
        hh = jnp.dot(xr[...], wr[...], preferred_element_type=jnp.float32)
        hs_ref[...] = hh * dinv[:, :1]

    return pl.pallas_call(
        body,
        grid=(n // _BR,),
        in_specs=[
            pl.BlockSpec((_BR, deg0.shape[1]), lambda i: (i, 0)),
            pl.BlockSpec((_BR, deg1.shape[1]), lambda i: (i, 0)),
            pl.BlockSpec((_BR, d), lambda i: (i, 0)),
            pl.BlockSpec((d, h), lambda i: (0, 0)),
        ],
        out_specs=[
            pl.BlockSpec((_BR, h), lambda i: (i, 0)),
            pl.BlockSpec((_BR, HW), lambda i: (i, 0)),
        ],
        out_shape=[
            jax.ShapeDtypeStruct((n, h), jnp.float32),
            jax.ShapeDtypeStruct((n, HW), jnp.float32),
        ],
    )(deg0, deg1, x, w)


def _tc_mid(p0, p1, dinv, b, w):
    n, d = p0.shape
    h = w.shape[1]

    def body(p0r, p1r, dvr, br, wr, out_ref):
        dv = dvr[:, :1]
        t = jnp.maximum((p0r[...] + p1r[...]) * dv + br[...], 0.0)
        out_ref[...] = jnp.dot(t, wr[...], preferred_element_type=jnp.float32) * dv

    return pl.pallas_call(
        body,
        grid=(n // _BR,),
        in_specs=[
            pl.BlockSpec((_BR, d), lambda i: (i, 0)),
            pl.BlockSpec((_BR, d), lambda i: (i, 0)),
            pl.BlockSpec((_BR, HW), lambda i: (i, 0)),
            pl.BlockSpec((1, d), lambda i: (0, 0)),
            pl.BlockSpec((d, h), lambda i: (0, 0)),
        ],
        out_specs=pl.BlockSpec((_BR, h), lambda i: (i, 0)),
        out_shape=jax.ShapeDtypeStruct((n, h), jnp.float32),
    )(p0, p1, dinv, b, w)


def _tc_scale(p0, p1, dinv, b):
    n, d = p0.shape

    def body(p0r, p1r, dvr, br, out_ref):
        dv = dvr[:, :1]
        out_ref[...] = jnp.maximum((p0r[...] + p1r[...]) * dv + br[...], 0.0) * dv

    return pl.pallas_call(
        body,
        grid=(n // _BR,),
        in_specs=[
            pl.BlockSpec((_BR, d), lambda i: (i, 0)),
            pl.BlockSpec((_BR, d), lambda i: (i, 0)),
            pl.BlockSpec((_BR, HW), lambda i: (i, 0)),
            pl.BlockSpec((1, d), lambda i: (0, 0)),
        ],
        out_specs=pl.BlockSpec((_BR, d), lambda i: (i, 0)),
        out_shape=jax.ShapeDtypeStruct((n, d), jnp.float32),
    )(p0, p1, dinv, b)


def _tc_final(p0, p1, dinv, w, b):
    n, d = p0.shape
    c = w.shape[1]

    def body(p0r, p1r, dvr, wr, br, out_ref):
        q = jnp.dot(p0r[...] + p1r[...], wr[...], preferred_element_type=jnp.float32)
        out_ref[...] = q * dvr[:, :1] + br[...]

    return pl.pallas_call(
        body,
        grid=(n // _BR,),
        in_specs=[
            pl.BlockSpec((_BR, d), lambda i: (i, 0)),
            pl.BlockSpec((_BR, d), lambda i: (i, 0)),
            pl.BlockSpec((_BR, HW), lambda i: (i, 0)),
            pl.BlockSpec((d, c), lambda i: (0, 0)),
            pl.BlockSpec((1, c), lambda i: (0, 0)),
        ],
        out_specs=pl.BlockSpec((_BR, c), lambda i: (i, 0)),
        out_shape=jax.ShapeDtypeStruct((n, c), jnp.float32),
    )(p0, p1, dinv, w, b)


def kernel(x, edge_index, batch, W1, b1, W2, b2, W3, b3):
    n, d = x.shape
    e = edge_index.shape[1]
    h = W1.shape[1]
    c = W3.shape[1]

    cap = NS * (CH0 + CH1) * L
    pad = cap - e - n
    loop = jnp.arange(n, dtype=jnp.int32)
    src = jnp.concatenate([edge_index[0], loop, jnp.zeros((pad,), jnp.int32)])
    dst = jnp.concatenate([edge_index[1], loop, jnp.full((pad,), n, jnp.int32)])
    packed_flat = (src << 16) | dst
    na = NS * CH0 * L
    part_a = packed_flat[:na].reshape(NS, CH0, L)
    part_b = packed_flat[na:].reshape(NS, CH1, L)
    part_b = jnp.concatenate(
        [part_b, jnp.zeros((NS, CH0 - CH1, L), jnp.int32)], axis=1)
    packed = jnp.concatenate([part_a, part_b], axis=0)

    onesf = jnp.ones((L, h), jnp.float32)
    zf = jnp.zeros((NACC, h), jnp.float32)

    deg = _make_hist(h)(packed, onesf, zf)
    hs1, dinv = _tc_first(deg[0:n], deg[NACC:NACC + n], x, W1)

    prop_h = _make_prop(h)
    p = prop_h(hs1, packed, zf)
    hs2 = _tc_mid(p[0:n], p[NACC:NACC + n], dinv, b1.reshape(1, -1), W2)

    p = prop_h(hs2, packed, zf)
    ts3 = _tc_scale(p[0:n], p[NACC:NACC + n], dinv, b2.reshape(1, -1))

    p3 = prop_h(ts3, packed, zf)
    return _tc_final(p3[0:n], p3[NACC:NACC + n], dinv, W3, b3.reshape(1, -1))
